# baseline (speedup 1.0000x reference)
import functools

import jax
import jax.numpy as jnp
from jax import lax
from jax.experimental import pallas as pl
from jax.experimental.pallas import tpu as pltpu
from jax.experimental.pallas import tpu_sc as plsc

N_EXP = 8
D = 768
N_TOK = 32768
NW = 32
TOK_PER_W = N_TOK // NW
LANES = 16
GROUPS = TOK_PER_W // LANES


def _proj_body(w_ref, x_ref, o_ref):
    o_ref[0] = lax.dot_general(
        w_ref[...], x_ref[...],
        dimension_numbers=(((1,), (1,)), ((), ())),
        preferred_element_type=jnp.float32,
    )


def _project(tokens, W):
    return pl.pallas_call(
        _proj_body,
        grid=(NW,),
        in_specs=[
            pl.BlockSpec((N_EXP, D), lambda i: (0, 0)),
            pl.BlockSpec((TOK_PER_W, D), lambda i: (i, 0)),
        ],
        out_specs=pl.BlockSpec((1, N_EXP, TOK_PER_W), lambda i: (i, 0, 0)),
        out_shape=jax.ShapeDtypeStruct((NW, N_EXP, TOK_PER_W), jnp.float32),
    )(W, tokens)


_mesh = plsc.VectorSubcoreMesh(core_axis_name="c", subcore_axis_name="s")


@functools.partial(
    pl.kernel,
    mesh=_mesh,
    out_type=[
        jax.ShapeDtypeStruct((N_TOK * 2,), jnp.float32),
        jax.ShapeDtypeStruct((N_TOK * 2,), jnp.int32),
    ],
    scratch_types=[
        pltpu.VMEM((N_EXP, TOK_PER_W), jnp.float32),
        pltpu.VMEM((TOK_PER_W * 2,), jnp.float32),
        pltpu.VMEM((TOK_PER_W * 2,), jnp.int32),
    ],
)
def _route(lg_hbm, sc_hbm, ix_hbm, lg_v, sc_v, ix_v):
    wid = lax.axis_index("s") * 2 + lax.axis_index("c")
    pltpu.sync_copy(lg_hbm.at[wid], lg_v)

    lanes = lax.iota(jnp.int32, LANES)
    even = (lanes & 1) == 0
    h0 = lanes >> 1
    h1 = h0 + 8

    def _vgather(x, idx):
        return lax.gather(
            x, idx.reshape(LANES, 1),
            lax.GatherDimensionNumbers(
                offset_dims=(), collapsed_slice_dims=(0,),
                start_index_map=(0,)),
            (1,), mode=lax.GatherScatterMode.PROMISE_IN_BOUNDS)

    def _interleave(a, b):
        lo = jnp.where(even, _vgather(a, h0), _vgather(b, h0))
        hi = jnp.where(even, _vgather(a, h1), _vgather(b, h1))
        return lo, hi

    def body(g, carry):
        base = g * LANES
        vs = [lg_v[e, pl.ds(base, LANES)] for e in range(N_EXP)]
        m = vs[0]
        for e in range(1, N_EXP):
            m = jnp.maximum(m, vs[e])
        ex = [jnp.exp(vs[e] - m) for e in range(N_EXP)]
        tot = ex[0]
        for e in range(1, N_EXP):
            tot = tot + ex[e]
        v1 = ex[0]
        i1 = jnp.zeros((LANES,), jnp.int32)
        for e in range(1, N_EXP):
            gt = ex[e] > v1
            v1 = jnp.where(gt, ex[e], v1)
            i1 = jnp.where(gt, jnp.int32(e), i1)
        v2 = jnp.full((LANES,), -1.0, jnp.float32)
        i2 = jnp.zeros((LANES,), jnp.int32)
        for e in range(N_EXP):
            ok = (ex[e] > v2) & (i1 != jnp.int32(e))
            v2 = jnp.where(ok, ex[e], v2)
            i2 = jnp.where(ok, jnp.int32(e), i2)
        s_lo, s_hi = _interleave(v1 / tot, v2 / tot)
        i_lo, i_hi = _interleave(i1, i2)
        fbase = base * 2
        sc_v[pl.ds(fbase, LANES)] = s_lo
        sc_v[pl.ds(fbase + LANES, LANES)] = s_hi
        ix_v[pl.ds(fbase, LANES)] = i_lo
        ix_v[pl.ds(fbase + LANES, LANES)] = i_hi
        return carry

    lax.fori_loop(0, GROUPS, body, 0)

    out0 = wid * (TOK_PER_W * 2)
    pltpu.sync_copy(sc_v, sc_hbm.at[pl.ds(out0, TOK_PER_W * 2)])
    pltpu.sync_copy(ix_v, ix_hbm.at[pl.ds(out0, TOK_PER_W * 2)])


def kernel(tokens, W):
    logits3 = _project(tokens, W)
    scores, idx = _route(logits3)
    return scores.reshape(N_TOK, 2), idx.reshape(N_TOK, 2)


# speedup vs baseline: 1.5975x; 1.5975x over previous
import jax
import jax.numpy as jnp
from jax import lax
from jax.experimental import pallas as pl

N_EXP = 8
D = 768
N_TOK = 32768
BLK = 1024
GRID = N_TOK // BLK


def _body(w_ref, x_ref, os_ref, oi_ref):
    lg = lax.dot_general(
        w_ref[...], x_ref[...],
        dimension_numbers=(((1,), (1,)), ((), ())),
        preferred_element_type=jnp.float32,
    )
    m = jnp.max(lg, axis=0, keepdims=True)
    ex = jnp.exp(lg - m)
    tot = jnp.sum(ex, axis=0, keepdims=True)
    rows = [ex[e:e + 1] for e in range(N_EXP)]
    v1 = rows[0]
    i1 = jnp.zeros((1, BLK), jnp.int32)
    for e in range(1, N_EXP):
        gt = rows[e] > v1
        v1 = jnp.where(gt, rows[e], v1)
        i1 = jnp.where(gt, jnp.int32(e), i1)
    v2 = jnp.full((1, BLK), -1.0, jnp.float32)
    i2 = jnp.zeros((1, BLK), jnp.int32)
    for e in range(N_EXP):
        ok = (rows[e] > v2) & (i1 != jnp.int32(e))
        v2 = jnp.where(ok, rows[e], v2)
        i2 = jnp.where(ok, jnp.int32(e), i2)
    s = jnp.concatenate([v1, v2], axis=0) / tot
    si = jnp.concatenate([i1, i2], axis=0)
    os_ref[...] = s.T
    oi_ref[...] = si.T


def kernel(tokens, W):
    return pl.pallas_call(
        _body,
        grid=(GRID,),
        in_specs=[
            pl.BlockSpec((N_EXP, D), lambda i: (0, 0)),
            pl.BlockSpec((BLK, D), lambda i: (i, 0)),
        ],
        out_specs=[
            pl.BlockSpec((BLK, 2), lambda i: (i, 0)),
            pl.BlockSpec((BLK, 2), lambda i: (i, 0)),
        ],
        out_shape=[
            jax.ShapeDtypeStruct((N_TOK, 2), jnp.float32),
            jax.ShapeDtypeStruct((N_TOK, 2), jnp.int32),
        ],
    )(W, tokens)


# speedup vs baseline: 3.1024x; 1.9420x over previous
import jax
import jax.numpy as jnp
from jax import lax
from jax.experimental import pallas as pl

N_EXP = 8
D = 768
N_TOK = 32768
BLK = 4096
GRID = N_TOK // BLK


def _body(w_ref, x_ref, os_ref, oi_ref):
    lg = lax.dot_general(
        w_ref[...], x_ref[...],
        dimension_numbers=(((1,), (1,)), ((), ())),
        preferred_element_type=jnp.float32,
    )
    m = jnp.max(lg, axis=0, keepdims=True)
    ex = jnp.exp(lg - m)
    tot = jnp.sum(ex, axis=0, keepdims=True)
    rows = [ex[e:e + 1] for e in range(N_EXP)]
    v1 = rows[0]
    i1 = jnp.zeros((1, BLK), jnp.int32)
    for e in range(1, N_EXP):
        gt = rows[e] > v1
        v1 = jnp.where(gt, rows[e], v1)
        i1 = jnp.where(gt, jnp.int32(e), i1)
    v2 = jnp.full((1, BLK), -1.0, jnp.float32)
    i2 = jnp.zeros((1, BLK), jnp.int32)
    for e in range(N_EXP):
        ok = (rows[e] > v2) & (i1 != jnp.int32(e))
        v2 = jnp.where(ok, rows[e], v2)
        i2 = jnp.where(ok, jnp.int32(e), i2)
    s = jnp.concatenate([v1, v2], axis=0) / tot
    si = jnp.concatenate([i1, i2], axis=0)
    os_ref[...] = s
    oi_ref[...] = si


def kernel(tokens, W):
    s, si = pl.pallas_call(
        _body,
        grid=(GRID,),
        in_specs=[
            pl.BlockSpec((N_EXP, D), lambda i: (0, 0)),
            pl.BlockSpec((BLK, D), lambda i: (i, 0)),
        ],
        out_specs=[
            pl.BlockSpec((2, BLK), lambda i: (0, i)),
            pl.BlockSpec((2, BLK), lambda i: (0, i)),
        ],
        out_shape=[
            jax.ShapeDtypeStruct((2, N_TOK), jnp.float32),
            jax.ShapeDtypeStruct((2, N_TOK), jnp.int32),
        ],
    )(W, tokens)
    return (jnp.stack([s[0], s[1]], axis=1),
            jnp.stack([si[0], si[1]], axis=1))


# speedup vs baseline: 3.5263x; 1.1366x over previous
import jax
import jax.numpy as jnp
from jax import lax
from jax.experimental import pallas as pl

N_EXP = 8
D = 768
N_TOK = 32768
BLK = 4096
GRID = N_TOK // BLK


def _body(w_ref, x_ref, os_ref, oi_ref):
    lg = lax.dot_general(
        w_ref[...], x_ref[...],
        dimension_numbers=(((1,), (1,)), ((), ())),
        preferred_element_type=jnp.float32,
    )
    m = jnp.max(lg, axis=0, keepdims=True)
    ex = jnp.exp(lg - m)
    tot = jnp.sum(ex, axis=0, keepdims=True)
    rows = [ex[e:e + 1] for e in range(N_EXP)]
    v1 = rows[0]
    i1 = jnp.zeros((1, BLK), jnp.int32)
    for e in range(1, N_EXP):
        gt = rows[e] > v1
        v1 = jnp.where(gt, rows[e], v1)
        i1 = jnp.where(gt, jnp.int32(e), i1)
    v2 = jnp.full((1, BLK), -1.0, jnp.float32)
    i2 = jnp.zeros((1, BLK), jnp.int32)
    for e in range(N_EXP):
        ok = (rows[e] > v2) & (i1 != jnp.int32(e))
        v2 = jnp.where(ok, rows[e], v2)
        i2 = jnp.where(ok, jnp.int32(e), i2)
    s = jnp.concatenate([v1, v2], axis=0) / tot
    si = jnp.concatenate([i1, i2], axis=0)
    os_ref[...] = s
    oi_ref[...] = si


def kernel(tokens, W):
    s, si = pl.pallas_call(
        _body,
        grid=(GRID,),
        in_specs=[
            pl.BlockSpec((N_EXP, D), lambda i: (0, 0)),
            pl.BlockSpec((BLK, D), lambda i: (i, 0)),
        ],
        out_specs=[
            pl.BlockSpec((2, BLK), lambda i: (0, i)),
            pl.BlockSpec((2, BLK), lambda i: (0, i)),
        ],
        out_shape=[
            jax.ShapeDtypeStruct((2, N_TOK), jnp.float32),
            jax.ShapeDtypeStruct((2, N_TOK), jnp.int32),
        ],
    )(W, tokens)
    return s.T, si.T
